# baseline (speedup 1.0000x reference)
import functools

import jax
import jax.numpy as jnp
from jax import lax
from jax.experimental import pallas as pl
from jax.experimental.pallas import tpu as pltpu
from jax.experimental.pallas import tpu_sc as plsc

BATCH = 4
SEQ_LEN = 2048
D_MODEL = 1024

NUM_CORES = 2
NUM_SUBCORES = 16
NUM_WORKERS = NUM_CORES * NUM_SUBCORES
SEQ_PER_W = SEQ_LEN // NUM_WORKERS
ROWS_PER_CHUNK = 32
CHUNKS = SEQ_PER_W // ROWS_PER_CHUNK
CHUNK_F = ROWS_PER_CHUNK * D_MODEL

_mesh = plsc.VectorSubcoreMesh(core_axis_name="c", subcore_axis_name="s")


@functools.partial(
    pl.kernel,
    mesh=_mesh,
    out_type=jax.ShapeDtypeStruct((BATCH * SEQ_LEN * D_MODEL,), jnp.float32),
    scratch_types=[
        pltpu.VMEM((CHUNK_F,), jnp.float32),
        pltpu.VMEM((CHUNK_F,), jnp.float32),
    ],
)
def _pos_enc_sc(x_hbm, emb_hbm, out_hbm, ebuf, xbuf):
    wid = lax.axis_index("s") * NUM_CORES + lax.axis_index("c")
    seq_base = wid * SEQ_PER_W
    for c in range(CHUNKS):
        eoff = (seq_base + c * ROWS_PER_CHUNK) * D_MODEL
        pltpu.sync_copy(emb_hbm.at[pl.ds(eoff, CHUNK_F)], ebuf)
        for b in range(BATCH):
            xoff = b * SEQ_LEN * D_MODEL + eoff
            pltpu.sync_copy(x_hbm.at[pl.ds(xoff, CHUNK_F)], xbuf)

            @plsc.parallel_loop(0, CHUNK_F, step=16, unroll=8)
            def _add(i):
                sl = pl.ds(i, 16)
                plsc.addupdate(xbuf.at[sl], ebuf[sl])

            pltpu.sync_copy(xbuf, out_hbm.at[pl.ds(xoff, CHUNK_F)])


def kernel(x, emb):
    out = _pos_enc_sc(x.reshape(-1), emb.reshape(-1))
    return out.reshape(BATCH, SEQ_LEN, D_MODEL)


# speedup vs baseline: 1.1370x; 1.1370x over previous
import functools

import jax
import jax.numpy as jnp
from jax import lax
from jax.experimental import pallas as pl
from jax.experimental.pallas import tpu as pltpu
from jax.experimental.pallas import tpu_sc as plsc

BATCH = 4
SEQ_LEN = 2048
D_MODEL = 1024

NUM_CORES = 2
NUM_SUBCORES = 16
NUM_WORKERS = NUM_CORES * NUM_SUBCORES
SEQ_PER_W = SEQ_LEN // NUM_WORKERS
ROWS_PER_CHUNK = 32
CHUNKS = SEQ_PER_W // ROWS_PER_CHUNK
CHUNK_F = ROWS_PER_CHUNK * D_MODEL

_mesh = plsc.VectorSubcoreMesh(core_axis_name="c", subcore_axis_name="s")


@functools.partial(
    pl.kernel,
    mesh=_mesh,
    out_type=jax.ShapeDtypeStruct((BATCH * SEQ_LEN * D_MODEL,), jnp.float32),
    scratch_types=[
        pltpu.VMEM((CHUNK_F,), jnp.float32),
        pltpu.VMEM((CHUNK_F,), jnp.float32),
        pltpu.VMEM((CHUNK_F,), jnp.float32),
        pltpu.SemaphoreType.DMA,
        pltpu.SemaphoreType.DMA,
        pltpu.SemaphoreType.DMA,
        pltpu.SemaphoreType.DMA,
        pltpu.SemaphoreType.DMA,
    ],
)
def _pos_enc_sc(x_hbm, emb_hbm, out_hbm, ebuf, xbuf0, xbuf1, esem, isem0, isem1, osem0, osem1):
    wid = lax.axis_index("s") * NUM_CORES + lax.axis_index("c")
    seq_base = wid * SEQ_PER_W
    xbufs = [xbuf0, xbuf1]
    isems = [isem0, isem1]
    osems = [osem0, osem1]
    for c in range(CHUNKS):
        eoff = (seq_base + c * ROWS_PER_CHUNK) * D_MODEL
        ecpy = pltpu.async_copy(emb_hbm.at[pl.ds(eoff, CHUNK_F)], ebuf, esem)
        incpy = [None] * BATCH
        outcpy = [None] * BATCH
        incpy[0] = pltpu.async_copy(
            x_hbm.at[pl.ds(0 * SEQ_LEN * D_MODEL + eoff, CHUNK_F)], xbufs[0], isems[0]
        )
        ecpy.wait()
        for b in range(BATCH):
            buf = xbufs[b % 2]
            incpy[b].wait()
            if b + 1 < BATCH:
                if b >= 1:
                    outcpy[b - 1].wait()
                incpy[b + 1] = pltpu.async_copy(
                    x_hbm.at[pl.ds((b + 1) * SEQ_LEN * D_MODEL + eoff, CHUNK_F)],
                    xbufs[(b + 1) % 2],
                    isems[(b + 1) % 2],
                )

            @plsc.parallel_loop(0, CHUNK_F, step=16, unroll=8)
            def _add(i, buf=buf):
                sl = pl.ds(i, 16)
                plsc.addupdate(buf.at[sl], ebuf[sl])

            outcpy[b] = pltpu.async_copy(
                buf, out_hbm.at[pl.ds(b * SEQ_LEN * D_MODEL + eoff, CHUNK_F)], osems[b % 2]
            )
        outcpy[BATCH - 2].wait()
        outcpy[BATCH - 1].wait()


def kernel(x, emb):
    out = _pos_enc_sc(x.reshape(-1), emb.reshape(-1))
    return out.reshape(BATCH, SEQ_LEN, D_MODEL)


# speedup vs baseline: 2.6975x; 2.3724x over previous
import functools

import jax
import jax.numpy as jnp
from jax import lax
from jax.experimental import pallas as pl
from jax.experimental.pallas import tpu as pltpu
from jax.experimental.pallas import tpu_sc as plsc

BATCH = 4
SEQ_LEN = 2048
D_MODEL = 1024

NUM_CORES = 2
NUM_SUBCORES = 16
NUM_WORKERS = NUM_CORES * NUM_SUBCORES
SEQ_PER_W = SEQ_LEN // NUM_WORKERS
ROWS_PER_CHUNK = 32
CHUNKS = SEQ_PER_W // ROWS_PER_CHUNK
VECS = ROWS_PER_CHUNK * D_MODEL // 16
LANES_PER_ROW = D_MODEL // 16

_mesh = plsc.VectorSubcoreMesh(core_axis_name="c", subcore_axis_name="s")


@functools.partial(
    pl.kernel,
    mesh=_mesh,
    out_type=jax.ShapeDtypeStruct((BATCH, SEQ_LEN, D_MODEL), jnp.float32),
    scratch_types=[
        pltpu.VMEM((ROWS_PER_CHUNK, D_MODEL), jnp.float32),
        pltpu.VMEM((ROWS_PER_CHUNK, D_MODEL), jnp.float32),
        pltpu.VMEM((ROWS_PER_CHUNK, D_MODEL), jnp.float32),
        pltpu.SemaphoreType.DMA,
        pltpu.SemaphoreType.DMA,
        pltpu.SemaphoreType.DMA,
        pltpu.SemaphoreType.DMA,
        pltpu.SemaphoreType.DMA,
    ],
)
def _pos_enc_sc(x_hbm, emb_hbm, out_hbm, ebuf, xbuf0, xbuf1, esem, isem0, isem1, osem0, osem1):
    wid = lax.axis_index("s") * NUM_CORES + lax.axis_index("c")
    seq_base = wid * SEQ_PER_W
    xbufs = [xbuf0, xbuf1]
    isems = [isem0, isem1]
    osems = [osem0, osem1]
    for c in range(CHUNKS):
        seq0 = seq_base + c * ROWS_PER_CHUNK
        ecpy = pltpu.async_copy(emb_hbm.at[pl.ds(seq0, ROWS_PER_CHUNK)], ebuf, esem)
        incpy = [None] * BATCH
        outcpy = [None] * BATCH
        incpy[0] = pltpu.async_copy(
            x_hbm.at[0, pl.ds(seq0, ROWS_PER_CHUNK)], xbufs[0], isems[0]
        )
        ecpy.wait()
        for b in range(BATCH):
            buf = xbufs[b % 2]
            incpy[b].wait()
            if b + 1 < BATCH:
                if b >= 1:
                    outcpy[b - 1].wait()
                incpy[b + 1] = pltpu.async_copy(
                    x_hbm.at[b + 1, pl.ds(seq0, ROWS_PER_CHUNK)],
                    xbufs[(b + 1) % 2],
                    isems[(b + 1) % 2],
                )

            @plsc.parallel_loop(0, VECS, step=1, unroll=8)
            def _add(i, buf=buf):
                r = i // LANES_PER_ROW
                col = (i % LANES_PER_ROW) * 16
                sl = pl.ds(col, 16)
                plsc.addupdate(buf.at[r, sl], ebuf[r, sl])

            outcpy[b] = pltpu.async_copy(
                buf, out_hbm.at[b, pl.ds(seq0, ROWS_PER_CHUNK)], osems[b % 2]
            )
        outcpy[BATCH - 2].wait()
        outcpy[BATCH - 1].wait()


def kernel(x, emb):
    return _pos_enc_sc(x, emb)


# speedup vs baseline: 2.7762x; 1.0292x over previous
import functools

import jax
import jax.numpy as jnp
from jax import lax
from jax.experimental import pallas as pl
from jax.experimental.pallas import tpu as pltpu
from jax.experimental.pallas import tpu_sc as plsc

BATCH = 4
SEQ_LEN = 2048
D_MODEL = 1024

NUM_CORES = 2
NUM_SUBCORES = 16
NUM_WORKERS = NUM_CORES * NUM_SUBCORES
SEQ_PER_W = SEQ_LEN // NUM_WORKERS
ROWS_PER_CHUNK = 16
CHUNKS = SEQ_PER_W // ROWS_PER_CHUNK
NBUF = 4
VECS = ROWS_PER_CHUNK * D_MODEL // 16
LANES_PER_ROW = D_MODEL // 16
JOBS = CHUNKS * BATCH

_mesh = plsc.VectorSubcoreMesh(core_axis_name="c", subcore_axis_name="s")

_scratch = (
    [pltpu.VMEM((ROWS_PER_CHUNK, D_MODEL), jnp.float32) for _ in range(2)]
    + [pltpu.VMEM((ROWS_PER_CHUNK, D_MODEL), jnp.float32) for _ in range(NBUF)]
    + [pltpu.SemaphoreType.DMA for _ in range(2 + 2 * NBUF)]
)


@functools.partial(
    pl.kernel,
    mesh=_mesh,
    out_type=jax.ShapeDtypeStruct((BATCH, SEQ_LEN, D_MODEL), jnp.float32),
    scratch_types=_scratch,
)
def _pos_enc_sc(x_hbm, emb_hbm, out_hbm, *bufs):
    ebufs = bufs[0:2]
    xbufs = bufs[2 : 2 + NBUF]
    esems = bufs[2 + NBUF : 4 + NBUF]
    isems = bufs[4 + NBUF : 4 + 2 * NBUF]
    osems = bufs[4 + 2 * NBUF : 4 + 3 * NBUF]

    wid = lax.axis_index("s") * NUM_CORES + lax.axis_index("c")
    seq_base = wid * SEQ_PER_W

    def seq0(c):
        return seq_base + c * ROWS_PER_CHUNK

    def issue_in(j):
        c, b = divmod(j, BATCH)
        return pltpu.async_copy(
            x_hbm.at[b, pl.ds(seq0(c), ROWS_PER_CHUNK)],
            xbufs[j % NBUF],
            isems[j % NBUF],
        )

    def issue_out(j):
        c, b = divmod(j, BATCH)
        return pltpu.async_copy(
            xbufs[j % NBUF],
            out_hbm.at[b, pl.ds(seq0(c), ROWS_PER_CHUNK)],
            osems[j % NBUF],
        )

    ecpys = [None] * CHUNKS
    incpy = [None] * JOBS
    outcpy = [None] * JOBS

    ecpys[0] = pltpu.async_copy(emb_hbm.at[pl.ds(seq0(0), ROWS_PER_CHUNK)], ebufs[0], esems[0])
    ecpys[1] = pltpu.async_copy(emb_hbm.at[pl.ds(seq0(1), ROWS_PER_CHUNK)], ebufs[1], esems[1])
    for j in range(NBUF - 1):
        incpy[j] = issue_in(j)

    for j in range(JOBS):
        c, b = divmod(j, BATCH)
        nj = j + NBUF - 1
        if nj < JOBS:
            if nj - NBUF >= 0:
                outcpy[nj - NBUF].wait()
            incpy[nj] = issue_in(nj)
        if b == 0:
            ecpys[c].wait()
        incpy[j].wait()

        ebuf = ebufs[c % 2]

        @plsc.parallel_loop(0, VECS, step=1, unroll=8)
        def _add(i, buf=xbufs[j % NBUF], ebuf=ebuf):
            r = i // LANES_PER_ROW
            col = (i % LANES_PER_ROW) * 16
            sl = pl.ds(col, 16)
            plsc.addupdate(buf.at[r, sl], ebuf[r, sl])

        if b == BATCH - 1 and c + 2 < CHUNKS:
            ecpys[c + 2] = pltpu.async_copy(
                emb_hbm.at[pl.ds(seq0(c + 2), ROWS_PER_CHUNK)],
                ebufs[(c + 2) % 2],
                esems[(c + 2) % 2],
            )

        outcpy[j] = issue_out(j)

    for j in range(JOBS - NBUF, JOBS):
        outcpy[j].wait()


def kernel(x, emb):
    return _pos_enc_sc(x, emb)
